# baseline (speedup 1.0000x reference)
import functools

import jax
import jax.numpy as jnp
from jax import lax
from jax.experimental import pallas as pl
from jax.experimental.pallas import tpu as pltpu
from jax.experimental.pallas import tpu_sc as plsc

N = 1_000_000
NW = 32
CHUNK = 31_248
TAIL = N - NW * CHUNK
TAIL_BASE = NW * CHUNK
LANES = 16


def _hc_vec(a, u):
    r = (1.0 - u) / u
    i = lax.bitcast_convert_type(r, jnp.int32)
    i = jnp.int32(0x5F3759DF) - lax.shift_right_logical(i, 1)
    y = lax.bitcast_convert_type(i, jnp.float32)
    y = y * (1.5 - 0.5 * r * y * y)
    y = y * (1.5 - 0.5 * r * y * y)
    y = y * (1.5 - 0.5 * r * y * y)
    q = r * r * y
    t = jnp.exp(-1.5 * a)
    s = 1.0 / (1.0 + t * q)
    return jnp.clip(1.2 * s - 0.1, 0.0, 1.0)


_mesh = plsc.VectorSubcoreMesh(core_axis_name="c", subcore_axis_name="s")


@functools.partial(
    pl.kernel,
    mesh=_mesh,
    out_type=jax.ShapeDtypeStruct((N,), jnp.float32),
    scratch_types=[
        pltpu.VMEM((CHUNK,), jnp.float32),
        pltpu.VMEM((CHUNK,), jnp.float32),
        pltpu.VMEM((CHUNK,), jnp.float32),
        pltpu.VMEM((TAIL,), jnp.float32),
        pltpu.VMEM((TAIL,), jnp.float32),
        pltpu.VMEM((TAIL,), jnp.float32),
    ],
)
def _hc_kernel(a_hbm, u_hbm, o_hbm, a_v, u_v, o_v, at_v, ut_v, ot_v):
    wid = lax.axis_index("s") * 2 + lax.axis_index("c")
    base = wid * CHUNK
    pltpu.sync_copy(a_hbm.at[pl.ds(base, CHUNK)], a_v)
    pltpu.sync_copy(u_hbm.at[pl.ds(base, CHUNK)], u_v)

    def body(j, carry):
        sl = pl.ds(j * LANES, LANES)
        o_v[sl] = _hc_vec(a_v[sl], u_v[sl])
        return carry

    lax.fori_loop(0, CHUNK // LANES, body, 0)
    pltpu.sync_copy(o_v, o_hbm.at[pl.ds(base, CHUNK)])

    @pl.when(wid == 0)
    def _tail():
        pltpu.sync_copy(a_hbm.at[pl.ds(TAIL_BASE, TAIL)], at_v)
        pltpu.sync_copy(u_hbm.at[pl.ds(TAIL_BASE, TAIL)], ut_v)
        for j in range(TAIL // LANES):
            sl = pl.ds(j * LANES, LANES)
            ot_v[sl] = _hc_vec(at_v[sl], ut_v[sl])
        pltpu.sync_copy(ot_v, o_hbm.at[pl.ds(TAIL_BASE, TAIL)])


def kernel(log_alpha, u, current_iter):
    return _hc_kernel(log_alpha, u)


# speedup vs baseline: 1.0362x; 1.0362x over previous
import functools

import jax
import jax.numpy as jnp
from jax import lax
from jax.experimental import pallas as pl
from jax.experimental.pallas import tpu as pltpu
from jax.experimental.pallas import tpu_sc as plsc

N = 1_000_000
NW = 32
CHUNK = 31_248
TAIL = N - NW * CHUNK
TAIL_BASE = NW * CHUNK
LANES = 16


def _hc_vec(a, u):
    g = jnp.exp(-a) * (1.0 - u) / u
    i = lax.bitcast_convert_type(g, jnp.int32)
    i = jnp.int32(0x5F3759DF) - lax.shift_right_logical(i, 1)
    y = lax.bitcast_convert_type(i, jnp.float32)
    y = y * (1.5 - 0.5 * g * y * y)
    y = y * (1.5 - 0.5 * g * y * y)
    z = g * g * y
    s = (1.1 - 0.1 * z) / (1.0 + z)
    return jnp.clip(s, 0.0, 1.0)


_mesh = plsc.VectorSubcoreMesh(core_axis_name="c", subcore_axis_name="s")


@functools.partial(
    pl.kernel,
    mesh=_mesh,
    out_type=jax.ShapeDtypeStruct((N,), jnp.float32),
    scratch_types=[
        pltpu.VMEM((CHUNK,), jnp.float32),
        pltpu.VMEM((CHUNK,), jnp.float32),
        pltpu.VMEM((CHUNK,), jnp.float32),
        pltpu.VMEM((TAIL,), jnp.float32),
        pltpu.VMEM((TAIL,), jnp.float32),
        pltpu.VMEM((TAIL,), jnp.float32),
    ],
)
def _hc_kernel(a_hbm, u_hbm, o_hbm, a_v, u_v, o_v, at_v, ut_v, ot_v):
    wid = lax.axis_index("s") * 2 + lax.axis_index("c")
    base = wid * CHUNK
    pltpu.sync_copy(a_hbm.at[pl.ds(base, CHUNK)], a_v)
    pltpu.sync_copy(u_hbm.at[pl.ds(base, CHUNK)], u_v)

    @plsc.parallel_loop(0, CHUNK, step=LANES, unroll=8)
    def _compute(i):
        sl = pl.ds(i, LANES)
        o_v[sl] = _hc_vec(a_v[sl], u_v[sl])
    pltpu.sync_copy(o_v, o_hbm.at[pl.ds(base, CHUNK)])

    @pl.when(wid == 0)
    def _tail():
        pltpu.sync_copy(a_hbm.at[pl.ds(TAIL_BASE, TAIL)], at_v)
        pltpu.sync_copy(u_hbm.at[pl.ds(TAIL_BASE, TAIL)], ut_v)
        for j in range(TAIL // LANES):
            sl = pl.ds(j * LANES, LANES)
            ot_v[sl] = _hc_vec(at_v[sl], ut_v[sl])
        pltpu.sync_copy(ot_v, o_hbm.at[pl.ds(TAIL_BASE, TAIL)])


def kernel(log_alpha, u, current_iter):
    return _hc_kernel(log_alpha, u)


# speedup vs baseline: 1.0516x; 1.0148x over previous
import functools

import jax
import jax.numpy as jnp
from jax import lax
from jax.experimental import pallas as pl
from jax.experimental.pallas import tpu as pltpu
from jax.experimental.pallas import tpu_sc as plsc

N = 1_000_000
NW = 32
CHUNK = 31_248
TAIL = N - NW * CHUNK
TAIL_BASE = NW * CHUNK
LANES = 16


def _hc_vec(a, u):
    g = jnp.exp(-a) * (1.0 - u) / u
    i = lax.bitcast_convert_type(g, jnp.int32)
    i = jnp.int32(0x5F3759DF) - lax.shift_right_logical(i, 1)
    y = lax.bitcast_convert_type(i, jnp.float32)
    y = y * (1.5 - 0.5 * g * y * y)
    y = y * (1.5 - 0.5 * g * y * y)
    z = g * g * y
    s = (1.1 - 0.1 * z) / (1.0 + z)
    return jnp.clip(s, 0.0, 1.0)


_mesh = plsc.VectorSubcoreMesh(core_axis_name="c", subcore_axis_name="s")


@functools.partial(
    pl.kernel,
    mesh=_mesh,
    out_type=jax.ShapeDtypeStruct((N,), jnp.float32),
    compiler_params=pltpu.CompilerParams(use_tc_tiling_on_sc=False),
    scratch_types=[
        pltpu.VMEM((CHUNK,), jnp.float32),
        pltpu.VMEM((CHUNK,), jnp.float32),
        pltpu.VMEM((CHUNK,), jnp.float32),
        pltpu.VMEM((TAIL,), jnp.float32),
        pltpu.VMEM((TAIL,), jnp.float32),
        pltpu.VMEM((TAIL,), jnp.float32),
    ],
)
def _hc_kernel(a_hbm, u_hbm, o_hbm, a_v, u_v, o_v, at_v, ut_v, ot_v):
    wid = lax.axis_index("s") * 2 + lax.axis_index("c")
    base = wid * CHUNK
    pltpu.sync_copy(a_hbm.at[pl.ds(base, CHUNK)], a_v)
    pltpu.sync_copy(u_hbm.at[pl.ds(base, CHUNK)], u_v)

    @plsc.parallel_loop(0, CHUNK, step=LANES, unroll=8)
    def _compute(i):
        sl = pl.ds(i, LANES)
        o_v[sl] = _hc_vec(a_v[sl], u_v[sl])
    pltpu.sync_copy(o_v, o_hbm.at[pl.ds(base, CHUNK)])

    @pl.when(wid == 0)
    def _tail():
        pltpu.sync_copy(a_hbm.at[pl.ds(TAIL_BASE, TAIL)], at_v)
        pltpu.sync_copy(u_hbm.at[pl.ds(TAIL_BASE, TAIL)], ut_v)
        for j in range(TAIL // LANES):
            sl = pl.ds(j * LANES, LANES)
            ot_v[sl] = _hc_vec(at_v[sl], ut_v[sl])
        pltpu.sync_copy(ot_v, o_hbm.at[pl.ds(TAIL_BASE, TAIL)])


def kernel(log_alpha, u, current_iter):
    return _hc_kernel(log_alpha, u)
